# baseline (speedup 1.0000x reference)
import functools

import jax
import jax.numpy as jnp
from jax import lax
from jax.experimental import pallas as pl
from jax.experimental.pallas import tpu as pltpu
from jax.experimental.pallas import tpu_sc as plsc

N = 10000
D_IN = 512
F = 128
E = 320000
NSUB = 16
CH = 128
NCH = 160
EPW = NCH * CH
EPAD = EPW * NSUB
NROWS = 10016
DEGROWS = 10240

_DN = (((1,), (1,)), ((), ()))


def _mesh():
    return plsc.VectorSubcoreMesh(core_axis_name="c", subcore_axis_name="s",
                                  num_cores=2)


def _deg_body(dst2_hbm, out_hbm, dst_all, ones_v, zbuf_v, deg_sh, sem_i,
              sem_s):
    cid = lax.axis_index("c")
    sid = lax.axis_index("s")
    wid = cid * NSUB + sid
    idx_src = dst2_hbm.at[pl.ds(wid * NCH, NCH)]
    pltpu.async_copy(idx_src, dst_all, sem_i)
    for i in range(CH // 16):
        ones_v[pl.ds(i * 16, 16)] = jnp.ones((16,), jnp.float32)
    for i in range(640 // 16):
        zbuf_v[pl.ds(i * 16, 16)] = jnp.zeros((16,), jnp.float32)
    pltpu.sync_copy(zbuf_v, deg_sh.at[pl.ds(sid * 640, 640)])
    pltpu.make_async_copy(idx_src, dst_all, sem_i).wait()
    plsc.subcore_barrier()

    def chunk(i, carry):
        for j in range(8):
            pltpu.async_copy(ones_v, deg_sh.at[dst_all.at[i * 8 + j]],
                             sem_s, add=True)
        for j in range(8):
            pltpu.make_async_copy(ones_v, deg_sh.at[dst_all.at[i * 8 + j]],
                                  sem_s).wait()
        return carry

    lax.fori_loop(0, NCH // 8, chunk, 0)
    plsc.subcore_barrier()
    pltpu.sync_copy(deg_sh.at[pl.ds(sid * 640, 640)],
                    out_hbm.at[pl.ds(cid * DEGROWS + sid * 640, 640)])


@functools.cache
def _deg_kernel():
    return pl.kernel(
        _deg_body,
        mesh=_mesh(),
        out_type=jax.ShapeDtypeStruct((2 * DEGROWS,), jnp.float32),
        scratch_types=[
            pltpu.VMEM((NCH, CH), jnp.int32),
            pltpu.VMEM((CH,), jnp.float32),
            pltpu.VMEM((640,), jnp.float32),
            pltpu.VMEM_SHARED((DEGROWS,), jnp.float32),
            pltpu.SemaphoreType.DMA,
            pltpu.SemaphoreType.DMA,
        ],
    )


def _scatter_body(g_hbm, src_hbm, dst2_hbm, out_hbm,
                  src_b, dst_b, rows_v, acc_sh, sem_is, sem_id, sem_g):
    cid = lax.axis_index("c")
    sid = lax.axis_index("s")
    wid = cid * NSUB + sid
    rowbase = wid * NCH

    def idx_fetch(grp, q):
        pltpu.async_copy(
            src_hbm.at[pl.ds((rowbase + grp * 4) * CH, 4 * CH)],
            src_b.at[q], sem_is)
        pltpu.async_copy(dst2_hbm.at[pl.ds(rowbase + grp * 4, 4)],
                         dst_b.at[pl.ds(q * 4, 4)], sem_id)

    def idx_wait(grp, q):
        pltpu.make_async_copy(
            src_hbm.at[pl.ds((rowbase + grp * 4) * CH, 4 * CH)],
            src_b.at[q], sem_is).wait()
        pltpu.make_async_copy(dst2_hbm.at[pl.ds(rowbase + grp * 4, 4)],
                              dst_b.at[pl.ds(q * 4, 4)], sem_id).wait()

    def g_fire(q, jj, p):
        pltpu.async_copy(g_hbm.at[src_b.at[q, pl.ds(jj * CH, CH)]],
                         rows_v.at[pl.ds(p * CH, CH)], sem_g)

    def g_wait(q, jj, p):
        pltpu.make_async_copy(g_hbm.at[src_b.at[q, pl.ds(jj * CH, CH)]],
                              rows_v.at[pl.ds(p * CH, CH)], sem_g).wait()

    idx_fetch(0, 0)
    idx_fetch(1, 1)
    pltpu.sync_copy(g_hbm.at[pl.ds(cid * N + sid * 624, 624)],
                    acc_sh.at[pl.ds(sid * 624, 624)])

    @pl.when(sid == 0)
    def _init_tail():
        pltpu.sync_copy(g_hbm.at[pl.ds(cid * N + 9984, 16)],
                        acc_sh.at[pl.ds(9984, 16)])

    idx_wait(0, 0)
    g_fire(0, 0, 0)
    plsc.subcore_barrier()

    def blk(bi, carry):
        c0 = bi * 8
        for j in range(8):
            c = c0 + j
            p = j % 2
            q, jj = divmod(j, 4)

            @pl.when(c + 1 < NCH)
            def _next_gather():
                jn = j + 1
                if jn == 8:
                    idx_wait(bi * 2 + 2, 0)
                    g_fire(0, 0, 1 - p)
                else:
                    qn, jjn = divmod(jn, 4)
                    if jjn == 0:
                        idx_wait(bi * 2 + 1, 1)
                    g_fire(qn, jjn, 1 - p)

            g_wait(q, jj, p)
            pltpu.sync_copy(rows_v.at[pl.ds(p * CH, CH)],
                            acc_sh.at[dst_b.at[4 * q + jj]], add=True)

            if j == 3:
                @pl.when(c0 + 8 < NCH)
                def _refetch_a():
                    idx_fetch(bi * 2 + 2, 0)
            if j == 7:
                @pl.when(c0 + 12 < NCH)
                def _refetch_b():
                    idx_fetch(bi * 2 + 3, 1)

        return carry

    lax.fori_loop(0, NCH // 8, blk, 0)
    plsc.subcore_barrier()
    pltpu.sync_copy(acc_sh.at[pl.ds(sid * 624, 624)],
                    out_hbm.at[pl.ds(cid * N + sid * 624, 624)])

    @pl.when(sid == 0)
    def _out_tail():
        pltpu.sync_copy(acc_sh.at[pl.ds(9984, 16)],
                        out_hbm.at[pl.ds(cid * N + 9984, 16)])


@functools.cache
def _scatter_kernel():
    return pl.kernel(
        _scatter_body,
        mesh=_mesh(),
        out_type=jax.ShapeDtypeStruct((2 * N, F), jnp.float32),
        scratch_types=[
            pltpu.VMEM((2, 4 * CH), jnp.int32),
            pltpu.VMEM((8, CH), jnp.int32),
            pltpu.VMEM((2 * CH, F), jnp.float32),
            pltpu.VMEM_SHARED((NROWS, F), jnp.float32),
            pltpu.SemaphoreType.DMA,
            pltpu.SemaphoreType.DMA,
            pltpu.SemaphoreType.DMA,
        ],
    )


def _proj_body(drug_ref, target_ref, Wd_ref, bd_ref, Wt_ref, bt_ref,
               Wg0_ref, h2_ref):
    hd = jnp.maximum(
        lax.dot_general(drug_ref[...], Wd_ref[...], _DN,
                        preferred_element_type=jnp.float32) + bd_ref[...], 0.0)
    h2_ref[0] = lax.dot_general(hd, Wg0_ref[...], _DN,
                                preferred_element_type=jnp.float32)
    ht = jnp.maximum(
        lax.dot_general(target_ref[...], Wt_ref[...], _DN,
                        preferred_element_type=jnp.float32) + bt_ref[...], 0.0)
    h2_ref[1] = lax.dot_general(ht, Wg0_ref[...], _DN,
                                preferred_element_type=jnp.float32)


def _proj_call(drug, target, Wd, bd, Wt, bt, Wg0):
    B = 1000
    grid = (N // B,)
    return pl.pallas_call(
        _proj_body,
        grid=grid,
        in_specs=[
            pl.BlockSpec((B, D_IN), lambda j: (j, 0)),
            pl.BlockSpec((B, D_IN), lambda j: (j, 0)),
            pl.BlockSpec((F, D_IN), lambda j: (0, 0)),
            pl.BlockSpec((1, F), lambda j: (0, 0)),
            pl.BlockSpec((F, D_IN), lambda j: (0, 0)),
            pl.BlockSpec((1, F), lambda j: (0, 0)),
            pl.BlockSpec((F, F), lambda j: (0, 0)),
        ],
        out_specs=pl.BlockSpec((2, B, F), lambda j: (0, j, 0)),
        out_shape=jax.ShapeDtypeStruct((2, N, F), jnp.float32),
    )(drug, target, Wd, bd, Wt, bt, Wg0)


def _scale_body(h2_ref, deg_ref, g2_ref, dinv2_ref):
    dinv = lax.rsqrt(deg_ref[...] + 1.0)
    dinv2_ref[...] = dinv
    g2_ref[0] = dinv[:, 0:1] * h2_ref[0]
    g2_ref[1] = dinv[:, 1:2] * h2_ref[1]


def _scale_call(h2, deg2):
    B = 1000
    grid = (N // B,)
    return pl.pallas_call(
        _scale_body,
        grid=grid,
        in_specs=[
            pl.BlockSpec((2, B, F), lambda j: (0, j, 0)),
            pl.BlockSpec((B, 2), lambda j: (j, 0)),
        ],
        out_specs=[
            pl.BlockSpec((2, B, F), lambda j: (0, j, 0)),
            pl.BlockSpec((B, 2), lambda j: (j, 0)),
        ],
        out_shape=[
            jax.ShapeDtypeStruct((2, N, F), jnp.float32),
            jax.ShapeDtypeStruct((N, 2), jnp.float32),
        ],
    )(h2, deg2)


def _mid_body(acc_ref, dinv_ref, Wg1_ref, bg0_ref, g2_ref):
    dinv = dinv_ref[...]
    for g in range(2):
        dv = dinv[:, g:g + 1]
        x1 = dv * acc_ref[g] + bg0_ref[...]
        h1 = lax.dot_general(x1, Wg1_ref[...], _DN,
                             preferred_element_type=jnp.float32)
        g2_ref[g] = dv * h1


def _mid_call(acc2, dinv2, Wg1, bg0):
    B = 1000
    grid = (N // B,)
    return pl.pallas_call(
        _mid_body,
        grid=grid,
        in_specs=[
            pl.BlockSpec((2, B, F), lambda j: (0, j, 0)),
            pl.BlockSpec((B, 2), lambda j: (j, 0)),
            pl.BlockSpec((F, F), lambda j: (0, 0)),
            pl.BlockSpec((1, F), lambda j: (0, 0)),
        ],
        out_specs=pl.BlockSpec((2, B, F), lambda j: (0, j, 0)),
        out_shape=jax.ShapeDtypeStruct((2, N, F), jnp.float32),
    )(acc2, dinv2, Wg1, bg0)


def _final_body(acc_ref, dinv_ref, bg1_ref, out_ref):
    B = acc_ref.shape[1]
    dinv = dinv_ref[...]
    dp = dinv[:, 0:1] * acc_ref[0] + bg1_ref[...]
    tp = dinv[:, 1:2] * acc_ref[1] + bg1_ref[...]
    dn = jnp.maximum(jnp.sqrt(jnp.sum(dp * dp, axis=1)), 1e-8)
    tn = jnp.maximum(jnp.sqrt(jnp.sum(tp * tp, axis=1)), 1e-8)
    cos = jnp.sum(dp * tp, axis=1) / (dn * tn)
    out_ref[...] = jax.nn.sigmoid(cos)[:, None]


def _final_call(acc2, dinv2, bg1):
    B = 1000
    grid = (N // B,)
    return pl.pallas_call(
        _final_body,
        grid=grid,
        in_specs=[
            pl.BlockSpec((2, B, F), lambda j: (0, j, 0)),
            pl.BlockSpec((B, 2), lambda j: (j, 0)),
            pl.BlockSpec((1, F), lambda j: (0, 0)),
        ],
        out_specs=pl.BlockSpec((B, 1), lambda j: (j, 0)),
        out_shape=jax.ShapeDtypeStruct((N, 1), jnp.float32),
    )(acc2, dinv2, bg1)


def kernel(drug, target, drug_edge_index, target_edge_index,
           Wd, bd, Wt, bt, Wg0, bg0, Wg1, bg1):
    pad = EPAD - E
    ar = jnp.arange(pad, dtype=jnp.int32)
    pad_src = ar % N
    pad_dst = N + (ar % 16)
    src_flat = jnp.concatenate([
        drug_edge_index[0], pad_src,
        target_edge_index[0] + N, pad_src + N,
    ])
    dst2 = jnp.concatenate([
        drug_edge_index[1], pad_dst,
        target_edge_index[1], pad_dst,
    ]).reshape(-1, CH)

    deg2 = _deg_kernel()(dst2).reshape(2, DEGROWS)[:, :N].T
    h2 = _proj_call(drug, target, Wd, bd.reshape(1, F), Wt, bt.reshape(1, F),
                    Wg0)
    g2, dinv2 = _scale_call(h2, deg2)
    acc = _scatter_kernel()(g2.reshape(2 * N, F), src_flat, dst2)
    g2b = _mid_call(acc.reshape(2, N, F), dinv2, Wg1, bg0.reshape(1, F))
    accb = _scatter_kernel()(g2b.reshape(2 * N, F), src_flat, dst2)
    out = _final_call(accb.reshape(2, N, F), dinv2, bg1.reshape(1, F))
    return out.reshape(N)


# speedup vs baseline: 1.0013x; 1.0013x over previous
import functools

import jax
import jax.numpy as jnp
from jax import lax
from jax.experimental import pallas as pl
from jax.experimental.pallas import tpu as pltpu
from jax.experimental.pallas import tpu_sc as plsc

N = 10000
D_IN = 512
F = 128
E = 320000
NSUB = 16
CH = 128
NCH = 160
EPW = NCH * CH
EPAD = EPW * NSUB
NROWS = 10016
DEGROWS = 10240

_DN = (((1,), (1,)), ((), ()))


def _mesh():
    return plsc.VectorSubcoreMesh(core_axis_name="c", subcore_axis_name="s",
                                  num_cores=2)


def _deg_body(dst2_hbm, out_hbm, dst_all, ones_v, zbuf_v, deg_sh, sem_i,
              sem_s):
    cid = lax.axis_index("c")
    sid = lax.axis_index("s")
    wid = cid * NSUB + sid
    idx_src = dst2_hbm.at[pl.ds(wid * NCH, NCH)]
    pltpu.async_copy(idx_src, dst_all, sem_i)
    for i in range(CH // 16):
        ones_v[pl.ds(i * 16, 16)] = jnp.ones((16,), jnp.float32)
    for i in range(640 // 16):
        zbuf_v[pl.ds(i * 16, 16)] = jnp.zeros((16,), jnp.float32)
    pltpu.sync_copy(zbuf_v, deg_sh.at[pl.ds(sid * 640, 640)])
    pltpu.make_async_copy(idx_src, dst_all, sem_i).wait()
    plsc.subcore_barrier()

    def chunk(i, carry):
        for j in range(8):
            pltpu.async_copy(ones_v, deg_sh.at[dst_all.at[i * 8 + j]],
                             sem_s, add=True)
        for j in range(8):
            pltpu.make_async_copy(ones_v, deg_sh.at[dst_all.at[i * 8 + j]],
                                  sem_s).wait()
        return carry

    lax.fori_loop(0, NCH // 8, chunk, 0)
    plsc.subcore_barrier()
    pltpu.sync_copy(deg_sh.at[pl.ds(sid * 640, 640)],
                    out_hbm.at[pl.ds(cid * DEGROWS + sid * 640, 640)])


@functools.cache
def _deg_kernel():
    return pl.kernel(
        _deg_body,
        mesh=_mesh(),
        out_type=jax.ShapeDtypeStruct((2 * DEGROWS,), jnp.float32),
        scratch_types=[
            pltpu.VMEM((NCH, CH), jnp.int32),
            pltpu.VMEM((CH,), jnp.float32),
            pltpu.VMEM((640,), jnp.float32),
            pltpu.VMEM_SHARED((DEGROWS,), jnp.float32),
            pltpu.SemaphoreType.DMA,
            pltpu.SemaphoreType.DMA,
        ],
    )


def _scatter_body(g_hbm, src_hbm, dst2_hbm, out_hbm,
                  src_b, dst_b, rows_v, acc_sh, sem_is, sem_id, sem_g):
    cid = lax.axis_index("c")
    sid = lax.axis_index("s")
    wid = cid * NSUB + sid
    rowbase = wid * NCH

    def idx_fetch(grp, q):
        pltpu.async_copy(
            src_hbm.at[pl.ds((rowbase + grp * 4) * CH, 4 * CH)],
            src_b.at[q], sem_is)
        pltpu.async_copy(dst2_hbm.at[pl.ds(rowbase + grp * 4, 4)],
                         dst_b.at[pl.ds(q * 4, 4)], sem_id)

    def idx_wait(grp, q):
        pltpu.make_async_copy(
            src_hbm.at[pl.ds((rowbase + grp * 4) * CH, 4 * CH)],
            src_b.at[q], sem_is).wait()
        pltpu.make_async_copy(dst2_hbm.at[pl.ds(rowbase + grp * 4, 4)],
                              dst_b.at[pl.ds(q * 4, 4)], sem_id).wait()

    def g_fire(q, jj, p):
        pltpu.async_copy(g_hbm.at[src_b.at[q, pl.ds(jj * CH, CH)]],
                         rows_v.at[pl.ds(p * CH, CH)], sem_g)

    def g_wait(q, jj, p):
        pltpu.make_async_copy(g_hbm.at[src_b.at[q, pl.ds(jj * CH, CH)]],
                              rows_v.at[pl.ds(p * CH, CH)], sem_g).wait()

    idx_fetch(0, 0)
    idx_fetch(1, 1)
    pltpu.sync_copy(g_hbm.at[pl.ds(cid * N + sid * 624, 624)],
                    acc_sh.at[pl.ds(sid * 624, 624)])

    @pl.when(sid == 0)
    def _init_tail():
        pltpu.sync_copy(g_hbm.at[pl.ds(cid * N + 9984, 16)],
                        acc_sh.at[pl.ds(9984, 16)])

    idx_wait(0, 0)
    g_fire(0, 0, 0)
    plsc.subcore_barrier()

    def blk(bi, carry):
        c0 = bi * 8
        for j in range(8):
            c = c0 + j
            p = j % 2
            q, jj = divmod(j, 4)

            @pl.when(c + 1 < NCH)
            def _next_gather():
                jn = j + 1
                if jn == 8:
                    idx_wait(bi * 2 + 2, 0)
                    g_fire(0, 0, 1 - p)
                else:
                    qn, jjn = divmod(jn, 4)
                    if jjn == 0:
                        idx_wait(bi * 2 + 1, 1)
                    g_fire(qn, jjn, 1 - p)

            g_wait(q, jj, p)
            pltpu.sync_copy(rows_v.at[pl.ds(p * CH, CH)],
                            acc_sh.at[dst_b.at[4 * q + jj]], add=True)

            if j == 3:
                @pl.when(c0 + 8 < NCH)
                def _refetch_a():
                    idx_fetch(bi * 2 + 2, 0)
            if j == 7:
                @pl.when(c0 + 12 < NCH)
                def _refetch_b():
                    idx_fetch(bi * 2 + 3, 1)

        return carry

    lax.fori_loop(0, NCH // 8, blk, 0)
    plsc.subcore_barrier()
    pltpu.sync_copy(acc_sh.at[pl.ds(sid * 624, 624)],
                    out_hbm.at[pl.ds(cid * N + sid * 624, 624)])

    @pl.when(sid == 0)
    def _out_tail():
        pltpu.sync_copy(acc_sh.at[pl.ds(9984, 16)],
                        out_hbm.at[pl.ds(cid * N + 9984, 16)])


@functools.cache
def _scatter_kernel():
    return pl.kernel(
        _scatter_body,
        mesh=_mesh(),
        out_type=jax.ShapeDtypeStruct((2 * N, F), jnp.float32),
        scratch_types=[
            pltpu.VMEM((2, 4 * CH), jnp.int32),
            pltpu.VMEM((8, CH), jnp.int32),
            pltpu.VMEM((2 * CH, F), jnp.float32),
            pltpu.VMEM_SHARED((NROWS, F), jnp.float32),
            pltpu.SemaphoreType.DMA,
            pltpu.SemaphoreType.DMA,
            pltpu.SemaphoreType.DMA,
        ],
    )


def _prep_body(drug_ref, target_ref, deg_ref, Wd_ref, bd_ref, Wt_ref, bt_ref,
               Wg0_ref, g2_ref, dinv2_ref):
    dinv = lax.rsqrt(deg_ref[...] + 1.0)
    dinv2_ref[...] = dinv
    hd = jnp.maximum(
        lax.dot_general(drug_ref[...], Wd_ref[...], _DN,
                        preferred_element_type=jnp.float32) + bd_ref[...], 0.0)
    g2_ref[0] = dinv[:, 0:1] * lax.dot_general(
        hd, Wg0_ref[...], _DN, preferred_element_type=jnp.float32)
    ht = jnp.maximum(
        lax.dot_general(target_ref[...], Wt_ref[...], _DN,
                        preferred_element_type=jnp.float32) + bt_ref[...], 0.0)
    g2_ref[1] = dinv[:, 1:2] * lax.dot_general(
        ht, Wg0_ref[...], _DN, preferred_element_type=jnp.float32)


def _prep_call(drug, target, deg2, Wd, bd, Wt, bt, Wg0):
    B = 1000
    grid = (N // B,)
    return pl.pallas_call(
        _prep_body,
        grid=grid,
        in_specs=[
            pl.BlockSpec((B, D_IN), lambda j: (j, 0)),
            pl.BlockSpec((B, D_IN), lambda j: (j, 0)),
            pl.BlockSpec((B, 2), lambda j: (j, 0)),
            pl.BlockSpec((F, D_IN), lambda j: (0, 0)),
            pl.BlockSpec((1, F), lambda j: (0, 0)),
            pl.BlockSpec((F, D_IN), lambda j: (0, 0)),
            pl.BlockSpec((1, F), lambda j: (0, 0)),
            pl.BlockSpec((F, F), lambda j: (0, 0)),
        ],
        out_specs=[
            pl.BlockSpec((2, B, F), lambda j: (0, j, 0)),
            pl.BlockSpec((B, 2), lambda j: (j, 0)),
        ],
        out_shape=[
            jax.ShapeDtypeStruct((2, N, F), jnp.float32),
            jax.ShapeDtypeStruct((N, 2), jnp.float32),
        ],
    )(drug, target, deg2, Wd, bd, Wt, bt, Wg0)


def _mid_body(acc_ref, dinv_ref, Wg1_ref, bg0_ref, g2_ref):
    dinv = dinv_ref[...]
    for g in range(2):
        dv = dinv[:, g:g + 1]
        x1 = dv * acc_ref[g] + bg0_ref[...]
        h1 = lax.dot_general(x1, Wg1_ref[...], _DN,
                             preferred_element_type=jnp.float32)
        g2_ref[g] = dv * h1


def _mid_call(acc2, dinv2, Wg1, bg0):
    B = 1000
    grid = (N // B,)
    return pl.pallas_call(
        _mid_body,
        grid=grid,
        in_specs=[
            pl.BlockSpec((2, B, F), lambda j: (0, j, 0)),
            pl.BlockSpec((B, 2), lambda j: (j, 0)),
            pl.BlockSpec((F, F), lambda j: (0, 0)),
            pl.BlockSpec((1, F), lambda j: (0, 0)),
        ],
        out_specs=pl.BlockSpec((2, B, F), lambda j: (0, j, 0)),
        out_shape=jax.ShapeDtypeStruct((2, N, F), jnp.float32),
    )(acc2, dinv2, Wg1, bg0)


def _final_body(acc_ref, dinv_ref, bg1_ref, out_ref):
    B = acc_ref.shape[1]
    dinv = dinv_ref[...]
    dp = dinv[:, 0:1] * acc_ref[0] + bg1_ref[...]
    tp = dinv[:, 1:2] * acc_ref[1] + bg1_ref[...]
    dn = jnp.maximum(jnp.sqrt(jnp.sum(dp * dp, axis=1)), 1e-8)
    tn = jnp.maximum(jnp.sqrt(jnp.sum(tp * tp, axis=1)), 1e-8)
    cos = jnp.sum(dp * tp, axis=1) / (dn * tn)
    out_ref[...] = jax.nn.sigmoid(cos)[:, None]


def _final_call(acc2, dinv2, bg1):
    B = 1000
    grid = (N // B,)
    return pl.pallas_call(
        _final_body,
        grid=grid,
        in_specs=[
            pl.BlockSpec((2, B, F), lambda j: (0, j, 0)),
            pl.BlockSpec((B, 2), lambda j: (j, 0)),
            pl.BlockSpec((1, F), lambda j: (0, 0)),
        ],
        out_specs=pl.BlockSpec((B, 1), lambda j: (j, 0)),
        out_shape=jax.ShapeDtypeStruct((N, 1), jnp.float32),
    )(acc2, dinv2, bg1)


def kernel(drug, target, drug_edge_index, target_edge_index,
           Wd, bd, Wt, bt, Wg0, bg0, Wg1, bg1):
    pad = EPAD - E
    ar = jnp.arange(pad, dtype=jnp.int32)
    pad_src = ar % N
    pad_dst = N + (ar % 16)
    src_flat = jnp.concatenate([
        drug_edge_index[0], pad_src,
        target_edge_index[0] + N, pad_src + N,
    ])
    dst2 = jnp.concatenate([
        drug_edge_index[1], pad_dst,
        target_edge_index[1], pad_dst,
    ]).reshape(-1, CH)

    deg2 = _deg_kernel()(dst2).reshape(2, DEGROWS)[:, :N].T
    g2, dinv2 = _prep_call(drug, target, deg2,
                           Wd, bd.reshape(1, F), Wt, bt.reshape(1, F), Wg0)
    acc = _scatter_kernel()(g2.reshape(2 * N, F), src_flat, dst2)
    g2b = _mid_call(acc.reshape(2, N, F), dinv2, Wg1, bg0.reshape(1, F))
    accb = _scatter_kernel()(g2b.reshape(2 * N, F), src_flat, dst2)
    out = _final_call(accb.reshape(2, N, F), dinv2, bg1.reshape(1, F))
    return out.reshape(N)
